# baseline (speedup 1.0000x reference)
import functools

import jax
import jax.numpy as jnp
from jax import lax
from jax.experimental import pallas as pl
from jax.experimental.pallas import tpu as pltpu

_N = 256
_K = 16
_B = 8
_BIGF = 1e9
_THRESH = 1e8
_NB = 64


def _shift_lane(x):
    return jnp.concatenate(
        [x[:, :, 1:], jnp.zeros((_B, 1, 1), jnp.float32)], axis=2)


def _shift_sublane(x):
    return jnp.concatenate(
        [x[:, 1:, :], jnp.zeros((_B, 1, 1), jnp.float32)], axis=1)


def _body(d_ref, c_ref, call_ref, ld_ref, cs_ref):
    bid = pl.program_id(0)

    @pl.when(bid == 0)
    def _phase1():
        D = d_ref[...]
        colB = lax.broadcasted_iota(jnp.int32, (_B, _N, _N), 2)
        rowB = lax.broadcasted_iota(jnp.int32, (_B, _N, _N), 1)
        ld_ref[...] = jnp.where(colB >= rowB, D, _BIGF)
        lowerD = ld_ref[...]
        upperT = jnp.where(colB <= rowB, jnp.transpose(D, (0, 2, 1)), _BIGF)
        col1 = lax.broadcasted_iota(jnp.int32, (_B, 1, _N), 2)
        row1 = lax.broadcasted_iota(jnp.int32, (_B, _N, 1), 1)

        c_cols = {0: D[:, :, _N - 1:_N]}
        c_rows = {0: upperT[:, _N - 1:_N, :]}
        H = _N // 2
        for kk in range(1, _K):
            limit = _N - kk
            if kk % 2 == 1:
                cs = jnp.where(col1 < limit, _shift_lane(c_rows[kk - 1]),
                               _BIGF)
                top = jnp.min(lowerD[:, :H, :] + cs, axis=2, keepdims=True)
                bot = jnp.min(lowerD[:, H:, H:] + cs[:, :, H:], axis=2,
                              keepdims=True)
                c_cols[kk] = jnp.concatenate([top, bot], axis=1)
            else:
                csc = jnp.where(row1 < limit, _shift_sublane(c_cols[kk - 1]),
                                _BIGF)
                left = jnp.min(upperT[:, :, :H] + csc, axis=1, keepdims=True)
                right = jnp.min(upperT[:, H:, H:] + csc[:, H:, :], axis=1,
                                keepdims=True)
                c_rows[kk] = jnp.concatenate([left, right], axis=2)

        cols_odd = jnp.concatenate([c_cols[kk] for kk in range(1, _K, 2)],
                                   axis=2)
        rows_odd = jnp.transpose(cols_odd, (0, 2, 1))
        for j, kk in enumerate(range(1, _K, 2)):
            c_rows[kk] = rows_odd[:, j:j + 1, :]
        crows = jnp.concatenate([c_rows[kk] for kk in range(_K)],
                                axis=1)

        sh = jnp.concatenate(
            [crows[:, :, 1:], jnp.zeros((_B, _K, 1), jnp.float32)], axis=2)
        shd = jnp.concatenate(
            [jnp.zeros((_B, 1, _N), jnp.float32), sh[:, :_K - 1, :]], axis=1)
        colM = lax.broadcasted_iota(jnp.int32, (_B, _K, _N), 2)
        kM = lax.broadcasted_iota(jnp.int32, (_B, _K, _N), 1)
        cs_ref[...] = jnp.where((kM >= 1) & (colM < _N - kM), shd, _BIGF)

        c_raw = jnp.transpose(crows, (0, 2, 1))
        rowC = lax.broadcasted_iota(jnp.int32, (_B, _N, _K), 1)
        kC = lax.broadcasted_iota(jnp.int32, (_B, _N, _K), 2)
        c_ref[...] = jnp.where((kC >= 1) & (rowC + kC >= _N), 0.0, c_raw)

    cs2 = cs_ref[bid]
    col16 = lax.broadcasted_iota(jnp.int32, (_K, _N), 1)
    kvec = lax.broadcasted_iota(jnp.int32, (_K, _N), 0)
    top_fix = (kvec == 0) & (col16 == _N - 1)

    _H = _N // 2

    def chunk(i, _):
        nn0 = i * _NB
        db = ld_ref[bid, pl.ds(nn0, _NB), :]
        a3 = db[:, None, :] + cs2[None, :, :]
        m = jnp.min(a3, axis=2, keepdims=True)
        e = jnp.exp(m - a3)
        r = 1.0 / jnp.sum(e, axis=2, keepdims=True)
        out = jnp.where(a3 < _THRESH, e * r, -1.0)
        out = jnp.where(top_fix[None], 1.0, out)
        call_ref[0, pl.ds(nn0, _NB), :, :] = out
        return 0

    def chunk_hi(i, _):
        nn0 = i * _NB
        db = ld_ref[bid, pl.ds(nn0, _NB), _H:]
        a3 = db[:, None, :] + cs2[None, :, _H:]
        m = jnp.min(a3, axis=2, keepdims=True)
        e = jnp.exp(m - a3)
        r = 1.0 / jnp.sum(e, axis=2, keepdims=True)
        out = jnp.where(a3 < _THRESH, e * r, -1.0)
        out = jnp.where(top_fix[None, :, _H:], 1.0, out)
        call_ref[0, pl.ds(nn0, _NB), :, :_H] = jnp.full(
            (_NB, _K, _H), -1.0, jnp.float32)
        call_ref[0, pl.ds(nn0, _NB), :, _H:] = out
        return 0

    lax.fori_loop(0, _H // _NB, chunk, 0)
    lax.fori_loop(_H // _NB, _N // _NB, chunk_hi, 0)


@jax.jit
def kernel(input_D_sum):
    return pl.pallas_call(
        _body,
        grid=(_B,),
        in_specs=[pl.BlockSpec((_B, _N, _N), lambda i: (0, 0, 0))],
        out_specs=[
            pl.BlockSpec((_B, _N, _K), lambda i: (0, 0, 0)),
            pl.BlockSpec((1, _N, _K, _N), lambda i: (i, 0, 0, 0)),
        ],
        out_shape=[
            jax.ShapeDtypeStruct((_B, _N, _K), jnp.float32),
            jax.ShapeDtypeStruct((_B, _N, _K, _N), jnp.float32),
        ],
        scratch_shapes=[
            pltpu.VMEM((_B, _N, _N), jnp.float32),
            pltpu.VMEM((_B, _K, _N), jnp.float32),
        ],
    )(input_D_sum)


# speedup vs baseline: 1.1135x; 1.1135x over previous
import functools

import jax
import jax.numpy as jnp
from jax import lax
from jax.experimental import pallas as pl
from jax.experimental.pallas import tpu as pltpu

_N = 256
_K = 16
_B = 8
_BIGF = 1e9
_THRESH = 1e8
_NB = 128


def _shift_lane(x):
    return jnp.concatenate(
        [x[:, :, 1:], jnp.zeros((_B, 1, 1), jnp.float32)], axis=2)


def _shift_sublane(x):
    return jnp.concatenate(
        [x[:, 1:, :], jnp.zeros((_B, 1, 1), jnp.float32)], axis=1)


def _body(d_ref, c_ref, call_ref, ld_ref, cs_ref):
    bid = pl.program_id(0)

    @pl.when(bid == 0)
    def _phase1():
        D = d_ref[...]
        colB = lax.broadcasted_iota(jnp.int32, (_B, _N, _N), 2)
        rowB = lax.broadcasted_iota(jnp.int32, (_B, _N, _N), 1)
        ld_ref[...] = jnp.where(colB >= rowB, D, _BIGF)
        lowerD = ld_ref[...]
        upperT = jnp.where(colB <= rowB, jnp.transpose(D, (0, 2, 1)), _BIGF)
        col1 = lax.broadcasted_iota(jnp.int32, (_B, 1, _N), 2)
        row1 = lax.broadcasted_iota(jnp.int32, (_B, _N, 1), 1)

        c_cols = {0: D[:, :, _N - 1:_N]}
        c_rows = {0: upperT[:, _N - 1:_N, :]}
        for kk in range(1, _K):
            limit = _N - kk
            if kk % 2 == 1:
                cs = jnp.where(col1 < limit, _shift_lane(c_rows[kk - 1]),
                               _BIGF)
                c_cols[kk] = jnp.min(lowerD + cs, axis=2, keepdims=True)
            else:
                csc = jnp.where(row1 < limit, _shift_sublane(c_cols[kk - 1]),
                                _BIGF)
                c_rows[kk] = jnp.min(upperT + csc, axis=1, keepdims=True)

        cols_odd = jnp.concatenate([c_cols[kk] for kk in range(1, _K, 2)],
                                   axis=2)
        rows_odd = jnp.transpose(cols_odd, (0, 2, 1))
        for j, kk in enumerate(range(1, _K, 2)):
            c_rows[kk] = rows_odd[:, j:j + 1, :]
        crows = jnp.concatenate([c_rows[kk] for kk in range(_K)],
                                axis=1)

        sh = jnp.concatenate(
            [crows[:, :, 1:], jnp.zeros((_B, _K, 1), jnp.float32)], axis=2)
        shd = jnp.concatenate(
            [jnp.zeros((_B, 1, _N), jnp.float32), sh[:, :_K - 1, :]], axis=1)
        colM = lax.broadcasted_iota(jnp.int32, (_B, _K, _N), 2)
        kM = lax.broadcasted_iota(jnp.int32, (_B, _K, _N), 1)
        cs_ref[...] = jnp.where((kM >= 1) & (colM < _N - kM), shd, _BIGF)

        c_raw = jnp.transpose(crows, (0, 2, 1))
        rowC = lax.broadcasted_iota(jnp.int32, (_B, _N, _K), 1)
        kC = lax.broadcasted_iota(jnp.int32, (_B, _N, _K), 2)
        c_ref[...] = jnp.where((kC >= 1) & (rowC + kC >= _N), 0.0, c_raw)

    cs2 = cs_ref[bid]
    col16 = lax.broadcasted_iota(jnp.int32, (_K, _N), 1)
    kvec = lax.broadcasted_iota(jnp.int32, (_K, _N), 0)
    top_fix = (kvec == 0) & (col16 == _N - 1)

    _H = _N // 2

    def chunk(i, _):
        nn0 = i * _NB
        db = ld_ref[bid, pl.ds(nn0, _NB), :]
        a3 = db[:, None, :] + cs2[None, :, :]
        m = jnp.min(a3, axis=2, keepdims=True)
        e = jnp.exp(m - a3)
        r = 1.0 / jnp.sum(e, axis=2, keepdims=True)
        out = jnp.where(a3 < _THRESH, e * r, -1.0)
        out = jnp.where(top_fix[None], 1.0, out)
        call_ref[0, pl.ds(nn0, _NB), :, :] = out
        return 0

    def chunk_hi(i, _):
        nn0 = i * _NB
        db = ld_ref[bid, pl.ds(nn0, _NB), _H:]
        a3 = db[:, None, :] + cs2[None, :, _H:]
        m = jnp.min(a3, axis=2, keepdims=True)
        e = jnp.exp(m - a3)
        r = 1.0 / jnp.sum(e, axis=2, keepdims=True)
        out = jnp.where(a3 < _THRESH, e * r, -1.0)
        out = jnp.where(top_fix[None, :, _H:], 1.0, out)
        call_ref[0, pl.ds(nn0, _NB), :, :_H] = jnp.full(
            (_NB, _K, _H), -1.0, jnp.float32)
        call_ref[0, pl.ds(nn0, _NB), :, _H:] = out
        return 0

    lax.fori_loop(0, _H // _NB, chunk, 0)
    lax.fori_loop(_H // _NB, _N // _NB, chunk_hi, 0)


@jax.jit
def kernel(input_D_sum):
    return pl.pallas_call(
        _body,
        grid=(_B,),
        in_specs=[pl.BlockSpec((_B, _N, _N), lambda i: (0, 0, 0))],
        out_specs=[
            pl.BlockSpec((_B, _N, _K), lambda i: (0, 0, 0)),
            pl.BlockSpec((1, _N, _K, _N), lambda i: (i, 0, 0, 0)),
        ],
        out_shape=[
            jax.ShapeDtypeStruct((_B, _N, _K), jnp.float32),
            jax.ShapeDtypeStruct((_B, _N, _K, _N), jnp.float32),
        ],
        scratch_shapes=[
            pltpu.VMEM((_B, _N, _N), jnp.float32),
            pltpu.VMEM((_B, _K, _N), jnp.float32),
        ],
    )(input_D_sum)
